# baseline (speedup 1.0000x reference)
import functools

import jax
import jax.numpy as jnp
from jax import lax
from jax.experimental import pallas as pl
from jax.experimental.pallas import tpu as pltpu
from jax.experimental.pallas import tpu_sc as plsc

N = 100000
D = 128
NW = 32
G = 200
NG = N // G
KMAX = -(-NG // NW)
NFULL = NG - NW * (KMAX - 1)
S1 = 104
S2 = G - S1

_mesh = plsc.VectorSubcoreMesh(core_axis_name="c", subcore_axis_name="s")


@functools.partial(
    pl.kernel,
    mesh=_mesh,
    out_type=jax.ShapeDtypeStruct((NG, G, D), jnp.float32),
    scratch_types=[
        pltpu.VMEM((KMAX * G,), jnp.int32),
        pltpu.VMEM((2, G, D), jnp.float32),
        pltpu.SemaphoreType.DMA,
        pltpu.SemaphoreType.DMA,
        pltpu.SemaphoreType.DMA,
        pltpu.SemaphoreType.DMA,
    ],
)
def _sc_embed(table_hbm, idx_hbm, out_hbm, idx_v, bufs, ga, gb, s0, s1):
    wid = lax.axis_index("s") * 2 + lax.axis_index("c")
    full = wid < NFULL
    a = jnp.where(full, KMAX * wid, (KMAX - 1) * wid + NFULL)
    ng = jnp.where(full, KMAX, KMAX - 1)

    base = pl.multiple_of(G * a, 8)
    pltpu.sync_copy(idx_hbm.at[pl.ds(base, (KMAX - 1) * G)],
                    idx_v.at[pl.ds(0, (KMAX - 1) * G)])
    last = pl.multiple_of(G * (a + ng - 1), 8)
    pltpu.sync_copy(idx_hbm.at[pl.ds(last, G)],
                    idx_v.at[pl.ds((KMAX - 1) * G, G)])

    ssems = (s0, s1)
    stores = [None, None]
    for k in range(KMAX):
        gk = jnp.minimum(k, ng - 1)
        b = k % 2
        if stores[b] is not None:
            stores[b].wait()
        iofs = pl.multiple_of(G * gk, 8)
        c1 = pltpu.async_copy(table_hbm.at[idx_v.at[pl.ds(iofs, S1)]],
                              bufs.at[b, pl.ds(0, S1)], ga)
        c2 = pltpu.async_copy(table_hbm.at[idx_v.at[pl.ds(iofs + S1, S2)]],
                              bufs.at[b, pl.ds(S1, S2)], gb)
        c1.wait()
        c2.wait()
        stores[b] = pltpu.async_copy(bufs.at[b], out_hbm.at[a + gk], ssems[b])
    stores[0].wait()
    stores[1].wait()


def kernel(x, enc_weight):
    idx = x.reshape(N).astype(jnp.int32)
    out = _sc_embed(enc_weight, idx)
    return out.reshape(N, D)


# speedup vs baseline: 1.0230x; 1.0230x over previous
import functools

import jax
import jax.numpy as jnp
from jax import lax
from jax.experimental import pallas as pl
from jax.experimental.pallas import tpu as pltpu
from jax.experimental.pallas import tpu_sc as plsc

N = 100000
D = 128
NW = 32
G = 200
NG = N // G
KMAX = -(-NG // NW)
NFULL = NG - NW * (KMAX - 1)
S1 = 104
S2 = G - S1

_mesh = plsc.VectorSubcoreMesh(core_axis_name="c", subcore_axis_name="s")


@functools.partial(
    pl.kernel,
    mesh=_mesh,
    out_type=jax.ShapeDtypeStruct((NG, G, D), jnp.float32),
    scratch_types=[
        pltpu.VMEM((KMAX * G,), jnp.int32),
        pltpu.VMEM((4, G, D), jnp.float32),
        pltpu.SemaphoreType.DMA,
        pltpu.SemaphoreType.DMA,
        pltpu.SemaphoreType.DMA,
        pltpu.SemaphoreType.DMA,
        pltpu.SemaphoreType.DMA,
        pltpu.SemaphoreType.DMA,
        pltpu.SemaphoreType.DMA,
        pltpu.SemaphoreType.DMA,
    ],
)
def _sc_embed(table_hbm, idx_hbm, out_hbm, idx_v, bufs,
              g0, g1, g2, g3, s0, s1, s2, s3):
    wid = lax.axis_index("s") * 2 + lax.axis_index("c")
    full = wid < NFULL
    a = jnp.where(full, KMAX * wid, (KMAX - 1) * wid + NFULL)
    ng = jnp.where(full, KMAX, KMAX - 1)

    base = pl.multiple_of(G * a, 8)
    pltpu.sync_copy(idx_hbm.at[pl.ds(base, (KMAX - 1) * G)],
                    idx_v.at[pl.ds(0, (KMAX - 1) * G)])
    last = pl.multiple_of(G * (a + ng - 1), 8)
    pltpu.sync_copy(idx_hbm.at[pl.ds(last, G)],
                    idx_v.at[pl.ds((KMAX - 1) * G, G)])

    NB = 4
    LAG = 2
    gsems = (g0, g1, g2, g3)
    ssems = (s0, s1, s2, s3)
    gathers = [None] * NB
    stores = [None] * NB

    def issue(k):
        b = k % NB
        if stores[b] is not None:
            stores[b].wait()
        gk = jnp.minimum(k, ng - 1)
        iofs = pl.multiple_of(G * gk, 8)
        c1 = pltpu.async_copy(table_hbm.at[idx_v.at[pl.ds(iofs, S1)]],
                              bufs.at[b, pl.ds(0, S1)], gsems[b])
        c2 = pltpu.async_copy(table_hbm.at[idx_v.at[pl.ds(iofs + S1, S2)]],
                              bufs.at[b, pl.ds(S1, S2)], gsems[b])
        gathers[b] = (c1, c2, a + gk)

    def drain(k):
        b = k % NB
        c1, c2, g = gathers[b]
        c1.wait()
        c2.wait()
        stores[b] = pltpu.async_copy(bufs.at[b], out_hbm.at[g], ssems[b])
        gathers[b] = None

    for k in range(KMAX):
        issue(k)
        if k >= LAG:
            drain(k - LAG)
    for k in range(KMAX - LAG, KMAX):
        drain(k)
    for st in stores:
        st.wait()


def kernel(x, enc_weight):
    idx = x.reshape(N).astype(jnp.int32)
    out = _sc_embed(enc_weight, idx)
    return out.reshape(N, D)


# speedup vs baseline: 2.3700x; 2.3168x over previous
import functools

import jax
import jax.numpy as jnp
from jax import lax
from jax.experimental import pallas as pl
from jax.experimental.pallas import tpu as pltpu
from jax.experimental.pallas import tpu_sc as plsc

N = 100000
D = 128
V = 21
NW = 32
G = 400
NG = N // G
KMAX = -(-NG // NW)
NFULL = NG - NW * (KMAX - 1)
NB = 2

_mesh = plsc.VectorSubcoreMesh(core_axis_name="c", subcore_axis_name="s")


@functools.partial(
    pl.kernel,
    mesh=_mesh,
    out_type=jax.ShapeDtypeStruct((NG, G, D), jnp.float32),
    scratch_types=[
        pltpu.VMEM((V, D), jnp.float32),
        pltpu.VMEM((NB, G, D), jnp.float32),
        pltpu.VMEM((KMAX * G,), jnp.int32),
        pltpu.SemaphoreType.DMA,
        pltpu.SemaphoreType.DMA,
    ],
)
def _sc_embed(table_hbm, idx_hbm, out_hbm, table_v, bufs, idx_v, s0, s1):
    wid = lax.axis_index("s") * 2 + lax.axis_index("c")
    full = wid < NFULL
    a = jnp.where(full, KMAX * wid, (KMAX - 1) * wid + NFULL)
    ng = jnp.where(full, KMAX, KMAX - 1)

    pltpu.sync_copy(table_hbm, table_v)

    base = pl.multiple_of(G * a, 8)
    pltpu.sync_copy(idx_hbm.at[pl.ds(base, (KMAX - 1) * G)],
                    idx_v.at[pl.ds(0, (KMAX - 1) * G)])
    last = pl.multiple_of(G * (a + ng - 1), 8)
    pltpu.sync_copy(idx_hbm.at[pl.ds(last, G)],
                    idx_v.at[pl.ds((KMAX - 1) * G, G)])

    ssems = (s0, s1)

    def group_of(k):
        gk = jnp.minimum(k, ng - 1)
        return a + gk

    stores = [None] * NB
    for k in range(KMAX):
        b = k % NB
        if stores[b] is not None:
            stores[b].wait()
        iofs = G * jnp.minimum(k, ng - 1)

        def copy_row(i, r):
            for j in range(8):
                sl = pl.ds(16 * j, 16)
                bufs[b, r, sl] = table_v[i, sl]

        def tile(t, _):
            iv = idx_v[pl.ds(iofs + 16 * t, 16)]
            for l in range(16):
                copy_row(iv[l], 16 * t + l)
            return 0

        lax.fori_loop(0, G // 16, tile, 0)
        stores[b] = pltpu.async_copy(bufs.at[b], out_hbm.at[group_of(k)],
                                     ssems[b])
    for st in stores:
        st.wait()


def kernel(x, enc_weight):
    idx = x.reshape(N).astype(jnp.int32)
    out = _sc_embed(enc_weight, idx)
    return out.reshape(N, D)


# speedup vs baseline: 5.0737x; 2.1407x over previous
import functools

import jax
import jax.numpy as jnp
from jax import lax
from jax.experimental import pallas as pl
from jax.experimental.pallas import tpu as pltpu
from jax.experimental.pallas import tpu_sc as plsc

N = 100000
D = 128
V = 21
NW = 32
G = 400
NG = N // G
KMAX = -(-NG // NW)
NFULL = NG - NW * (KMAX - 1)
NB = 2

_mesh = plsc.VectorSubcoreMesh(core_axis_name="c", subcore_axis_name="s")


@functools.partial(
    pl.kernel,
    mesh=_mesh,
    out_type=jax.ShapeDtypeStruct((NG, G, D), jnp.float32),
    scratch_types=[
        pltpu.VMEM((V, D), jnp.float32),
        pltpu.VMEM((NB, G, D), jnp.float32),
        pltpu.VMEM((KMAX * G,), jnp.int32),
        pltpu.SemaphoreType.DMA,
        pltpu.SemaphoreType.DMA,
    ],
)
def _sc_embed(table_hbm, idx_hbm, out_hbm, table_v, bufs, idx_v, s0, s1):
    wid = lax.axis_index("s") * 2 + lax.axis_index("c")
    full = wid < NFULL
    a = jnp.where(full, KMAX * wid, (KMAX - 1) * wid + NFULL)
    ng = jnp.where(full, KMAX, KMAX - 1)

    pltpu.sync_copy(table_hbm, table_v)

    base = pl.multiple_of(G * a, 8)
    pltpu.sync_copy(idx_hbm.at[pl.ds(base, (KMAX - 1) * G)],
                    idx_v.at[pl.ds(0, (KMAX - 1) * G)])
    last = pl.multiple_of(G * (a + ng - 1), 8)
    pltpu.sync_copy(idx_hbm.at[pl.ds(last, G)],
                    idx_v.at[pl.ds((KMAX - 1) * G, G)])

    ssems = (s0, s1)

    def group_of(k):
        gk = jnp.minimum(k, ng - 1)
        return a + gk

    stores = [None] * NB
    for k in range(KMAX):
        b = k % NB
        if stores[b] is not None:
            stores[b].wait()
        iofs = G * jnp.minimum(k, ng - 1)

        def copy_row(i, r):
            vals = [table_v[i, pl.ds(16 * j, 16)] for j in range(8)]
            for j, v in enumerate(vals):
                bufs[b, r, pl.ds(16 * j, 16)] = v

        def tile(t, _):
            iv = idx_v[pl.ds(iofs + 16 * t, 16)]
            lanes = [iv[l] for l in range(16)]
            for l in range(16):
                copy_row(lanes[l], 16 * t + l)
            return 0

        lax.fori_loop(0, G // 16, tile, 0)
        stores[b] = pltpu.async_copy(bufs.at[b], out_hbm.at[group_of(k)],
                                     ssems[b])
    for st in stores:
        st.wait()


def kernel(x, enc_weight):
    idx = x.reshape(N).astype(jnp.int32)
    out = _sc_embed(enc_weight, idx)
    return out.reshape(N, D)


# speedup vs baseline: 5.3332x; 1.0512x over previous
import functools

import jax
import jax.numpy as jnp
from jax import lax
from jax.experimental import pallas as pl
from jax.experimental.pallas import tpu as pltpu
from jax.experimental.pallas import tpu_sc as plsc

N = 100000
D = 128
V = 21
NW = 32
G = 400
NG = N // G
KMAX = -(-NG // NW)
NFULL = NG - NW * (KMAX - 1)
NB = 2

_mesh = plsc.VectorSubcoreMesh(core_axis_name="c", subcore_axis_name="s")


@functools.partial(
    pl.kernel,
    mesh=_mesh,
    out_type=jax.ShapeDtypeStruct((NG, G, D), jnp.float32),
    scratch_types=[
        pltpu.VMEM((V, D), jnp.float32),
        pltpu.VMEM((NB, G, D), jnp.float32),
        pltpu.VMEM((KMAX * G,), jnp.int32),
        pltpu.SemaphoreType.DMA,
        pltpu.SemaphoreType.DMA,
    ],
)
def _sc_embed(table_hbm, idx_hbm, out_hbm, table_v, bufs, idx_v, s0, s1):
    wid = lax.axis_index("s") * 2 + lax.axis_index("c")
    full = wid < NFULL
    a = jnp.where(full, KMAX * wid, (KMAX - 1) * wid + NFULL)
    ng = jnp.where(full, KMAX, KMAX - 1)

    pltpu.sync_copy(table_hbm, table_v)

    base = pl.multiple_of(G * a, 8)
    pltpu.sync_copy(idx_hbm.at[pl.ds(base, (KMAX - 1) * G)],
                    idx_v.at[pl.ds(0, (KMAX - 1) * G)])
    last = pl.multiple_of(G * (a + ng - 1), 8)
    pltpu.sync_copy(idx_hbm.at[pl.ds(last, G)],
                    idx_v.at[pl.ds((KMAX - 1) * G, G)])

    ssems = (s0, s1)

    def group_of(k):
        gk = jnp.minimum(k, ng - 1)
        return a + gk

    stores = [None] * NB
    for k in range(KMAX):
        b = k % NB
        if stores[b] is not None:
            stores[b].wait()
        iofs = G * jnp.minimum(k, ng - 1)

        def copy_row(i, r):
            vals = [table_v[i, pl.ds(16 * j, 16)] for j in range(8)]
            for j, v in enumerate(vals):
                bufs[b, r, pl.ds(16 * j, 16)] = v

        @plsc.parallel_loop(0, G // 16, 1, unroll=2)
        def tile(t):
            iv = idx_v[pl.ds(iofs + 16 * t, 16)]
            lanes = [iv[l] for l in range(16)]
            for l in range(16):
                copy_row(lanes[l], 16 * t + l)
        stores[b] = pltpu.async_copy(bufs.at[b], out_hbm.at[group_of(k)],
                                     ssems[b])
    for st in stores:
        st.wait()


def kernel(x, enc_weight):
    idx = x.reshape(N).astype(jnp.int32)
    out = _sc_embed(enc_weight, idx)
    return out.reshape(N, D)
